# baseline (speedup 1.0000x reference)
import functools

import jax
import jax.numpy as jnp
from jax import lax
from jax.experimental import pallas as pl
from jax.experimental.pallas import tpu as pltpu
from jax.experimental.pallas import tpu_sc as plsc

VOCAB_ = 1000000
DIM_ = 64
B_ = 16384
NNEG_ = 5
MARGIN_ = 1.0

NC = 2
NS = 16
NW = NC * NS
LANES = 16

NPOS = 7 * B_
SB = 512
NSB = 61
WRANGE = NSB * SB
TAIL = 7812 * 128
ECAP = 8192
RING = 8

CHUNK = 128
GROUPS = CHUNK // LANES
BPW = B_ // NW


def _shuf(x, perm):
  return lax.gather(
      x, perm[:, None],
      lax.GatherDimensionNumbers(offset_dims=(), collapsed_slice_dims=(0,),
                                 start_index_map=(0,)),
      slice_sizes=(1,), mode=lax.GatherScatterMode.PROMISE_IN_BOUNDS)


def _gather_body(u_hbm, v_hbm, n0_hbm, n1_hbm, n2_hbm, n3_hbm, n4_hbm,
                 srct_hbm, tgtt_hbm, stail_hbm, ttail_hbm, rows_hbm,
                 blk, blk2, tblk, ev, ep, sv, sp, ixb, rowstage, semb, semr):
  cid = lax.axis_index("c")
  sid = lax.axis_index("s")
  wid = cid * NS + sid
  lane = lax.iota(jnp.int32, LANES)

  lo = wid * WRANGE
  is_last = wid == NW - 1
  hi = jnp.where(is_last, VOCAB_, lo + WRANGE)
  nsb = jnp.where(is_last, NSB + 1, NSB)

  def sweep(tab_hbm, tail_hbm, idx_arrays, pos_base_list):
    def scan_round(r):
      def scan_arr(carry, arr, pbase):
        cur, seen = carry

        def chunk_body(c, carry2):
          cur2, seen2 = carry2
          pltpu.sync_copy(arr.at[pl.ds(c * 2048, 2048)], ixb)

          def vec_body(k, carry3):
            cur3, seen3 = carry3
            v = ixb[pl.ds(k * LANES, LANES)]
            m = (v >= lo) & (v < hi)
            call = plsc.all_reduce_population_count(m)[0]

            def append(args):
              cur4, seen4 = args
              mi = m.astype(jnp.int32)
              pref = plsc.cumsum(mi)
              rank = seen4 + pref - 1
              rlo = r * ECAP
              m2 = m & (rank >= rlo) & (rank < rlo + ECAP)
              posv = pbase + c * 2048 + k * LANES + lane
              plsc.store_compressed(ev.at[pl.ds(cur4, LANES)], v, mask=m2)
              plsc.store_compressed(ep.at[pl.ds(cur4, LANES)], posv, mask=m2)
              c2 = plsc.all_reduce_population_count(m2)[0]
              return cur4 + c2, seen4 + call

            return lax.cond(call > 0, append, lambda a: a, (cur3, seen3))

          return lax.fori_loop(0, 128, vec_body, (cur2, seen2),
                               unroll=2)

        return lax.fori_loop(0, B_ // 2048, chunk_body, (cur, seen))

      cur, seen = jnp.int32(0), jnp.int32(0)
      for arr, pbase in zip(idx_arrays, pos_base_list):
        cur, seen = scan_arr((cur, seen), arr, pbase)
      return cur, seen

    def extract_block(width, blo, nent, src_sel):
      src_buf = (blk, blk2, tblk)[src_sel]
      def cmp_body(j, scur):
        v = ev[pl.ds(j * LANES, LANES)]
        valid = (j * LANES + lane) < nent
        m = valid & (v >= blo) & (v < blo + width)
        cm = plsc.all_reduce_population_count(m)[0]

        def append(scur2):
          p = ep[pl.ds(j * LANES, LANES)]
          plsc.store_compressed(sv.at[pl.ds(scur2, LANES)], v, mask=m)
          plsc.store_compressed(sp.at[pl.ds(scur2, LANES)], p, mask=m)
          return scur2 + cm

        return lax.cond(cm > 0, append, lambda a: a, scur)

      nvec = (nent + LANES - 1) // LANES
      scnt = lax.fori_loop(0, nvec, cmp_body, jnp.int32(0))

      def ext_body(jj, carry):
        @pl.when(jj >= RING)
        def _():
          for _j in range(LANES):
            pltpu.make_async_copy(rows_hbm.at[pl.ds(0, DIM_)],
                                  rowstage.at[pl.ds(0, DIM_)], semr).wait()
        rbase = (jj % RING) * (LANES * DIM_)
        v16 = sv[pl.ds(jj * LANES, LANES)]
        p16 = sp[pl.ds(jj * LANES, LANES)]
        em = (jj * LANES + lane) < scnt
        col = jnp.where(em, v16 - blo, 0)
        posd = jnp.where(em, p16, NPOS + lane)
        for d in range(DIM_):
          if src_sel == 2:
            g = plsc.load_gather(src_buf,
                                 [col, jnp.full((LANES,), d, jnp.int32)])
          else:
            g = plsc.load_gather(src_buf,
                                 [jnp.full((LANES,), d, jnp.int32), col])
          plsc.store_scatter(rowstage, [rbase + lane * DIM_ + d], g)
        for j in range(LANES):
          pltpu.make_async_copy(
              rowstage.at[pl.ds(rbase + j * DIM_, DIM_)],
              rows_hbm.at[pl.ds(posd[j] * DIM_, DIM_)], semr).start()
        return carry

      nsvec = (scnt + LANES - 1) // LANES
      lax.fori_loop(0, nsvec, ext_body, jnp.int32(0))
      nrem = jnp.minimum(nsvec, RING) * LANES

      def tail_drain(_, c2):
        pltpu.make_async_copy(rows_hbm.at[pl.ds(0, DIM_)],
                              rowstage.at[pl.ds(0, DIM_)], semr).wait()
        return c2

      lax.fori_loop(0, nrem, tail_drain, jnp.int32(0))

    def round_body(carry):
      r, _total = carry
      nent, total = scan_round(r)

      def stage(i, buf):
        return pltpu.make_async_copy(
            tab_hbm.at[:, pl.ds(lo + i * SB, SB)], buf, semb)

      stage(0, blk).start()

      def pair_body(i2, c2):
        i = i2 * 2

        @pl.when(i < nsb)
        def _():
          stage(i, blk).wait()

          @pl.when(i + 1 < nsb)
          def _():
            stage(i + 1, blk2).start()
          extract_block(SB, lo + i * SB, nent, 0)

        @pl.when(i + 1 < nsb)
        def _():
          stage(i + 1, blk2).wait()

          @pl.when(i + 2 < nsb)
          def _():
            stage(i + 2, blk).start()
          extract_block(SB, lo + (i + 1) * SB, nent, 1)

        return c2

      lax.fori_loop(0, (NSB + 2) // 2, pair_body, jnp.int32(0))

      @pl.when(is_last)
      def _():
        pltpu.sync_copy(tail_hbm, tblk)
        extract_block(VOCAB_ - TAIL, jnp.int32(TAIL), nent, 2)

      return r + 1, total

    def round_cond(carry):
      r, total = carry
      return jnp.logical_or(r == 0, r * ECAP < total)

    lax.while_loop(round_cond, round_body, (jnp.int32(0), jnp.int32(0)))

  sweep(srct_hbm, stail_hbm, [u_hbm], [0])
  sweep(tgtt_hbm, ttail_hbm,
        [v_hbm, n0_hbm, n1_hbm, n2_hbm, n3_hbm, n4_hbm],
        [B_, 2 * B_, 3 * B_, 4 * B_, 5 * B_, 6 * B_])


def _compute_body(rows_hbm, out_hbm, ru, rv, rn0, rn1, rn2, rn3, rn4,
                  acc_v, sem):
  cid = lax.axis_index("c")
  sid = lax.axis_index("s")
  wid = cid * NS + sid
  lane = lax.iota(jnp.int32, LANES)
  perms = [lane ^ 1, lane ^ 2, lane ^ 4, lane ^ 8]
  total = jnp.zeros((LANES,), jnp.float32)

  for chunk in range(BPW // CHUNK):
    base = wid * BPW + chunk * CHUNK
    bufs = (ru, rv, rn0, rn1, rn2, rn3, rn4)
    cps = [pltpu.make_async_copy(
        rows_hbm.at[pl.ds((t * B_ + base) * DIM_, CHUNK * DIM_)], bufs[t], sem)
        for t in range(7)]
    for cp in cps:
      cp.start()
    for cp in cps:
      cp.wait()

    def elem_body(e, tot):
      pv = jnp.zeros((LANES,), jnp.float32)
      p0 = jnp.zeros((LANES,), jnp.float32)
      p1 = jnp.zeros((LANES,), jnp.float32)
      p2 = jnp.zeros((LANES,), jnp.float32)
      p3 = jnp.zeros((LANES,), jnp.float32)
      p4 = jnp.zeros((LANES,), jnp.float32)
      for k in range(DIM_ // LANES):
        sl = pl.ds(e * DIM_ + k * LANES, LANES)
        uc = ru[sl]
        pv = pv + uc * rv[sl]
        p0 = p0 + uc * rn0[sl]
        p1 = p1 + uc * rn1[sl]
        p2 = p2 + uc * rn2[sl]
        p3 = p3 + uc * rn3[sl]
        p4 = p4 + uc * rn4[sl]
      loss = jnp.zeros((LANES,), jnp.float32)
      for p in (p0, p1, p2, p3, p4):
        r = p - pv
        for perm in perms:
          r = r + _shuf(r, perm)
        loss = loss + jnp.maximum(r + MARGIN_, 0.0)
      return tot + loss

    total = total + lax.fori_loop(0, CHUNK, elem_body,
                                  jnp.zeros((LANES,), jnp.float32))

  acc_v[...] = jnp.where(lane == 0, total, jnp.float32(0.0))
  pltpu.sync_copy(acc_v, out_hbm.at[pl.ds(wid * LANES, LANES)])


@jax.jit
def _sc_call(u_pos, v_pos, n0, n1, n2, n3, n4, src_t, tgt_t, stail, ttail):
  mesh = plsc.VectorSubcoreMesh(core_axis_name="c", subcore_axis_name="s")
  cp = pltpu.CompilerParams(needs_layout_passes=False)
  gather = pl.kernel(
      _gather_body,
      out_type=jax.ShapeDtypeStruct(((NPOS + LANES) * DIM_,), jnp.float32),
      mesh=mesh,
      compiler_params=cp,
      scratch_types=[
          pltpu.VMEM((DIM_, SB), jnp.float32),
          pltpu.VMEM((DIM_, SB), jnp.float32),
          pltpu.VMEM((VOCAB_ - TAIL, DIM_), jnp.float32),
          pltpu.VMEM((ECAP + LANES, ), jnp.int32),
          pltpu.VMEM((ECAP + LANES, ), jnp.int32),
          pltpu.VMEM((ECAP + LANES, ), jnp.int32),
          pltpu.VMEM((ECAP + LANES, ), jnp.int32),
          pltpu.VMEM((2048,), jnp.int32),
          pltpu.VMEM((RING * LANES * DIM_,), jnp.float32),
          pltpu.SemaphoreType.DMA,
          pltpu.SemaphoreType.DMA,
      ],
  )
  rows = gather(u_pos, v_pos, n0, n1, n2, n3, n4, src_t, tgt_t, stail, ttail)
  compute = pl.kernel(
      _compute_body,
      out_type=jax.ShapeDtypeStruct((NW * LANES,), jnp.float32),
      mesh=mesh,
      compiler_params=cp,
      scratch_types=[
          pltpu.VMEM((CHUNK * DIM_,), jnp.float32),
          pltpu.VMEM((CHUNK * DIM_,), jnp.float32),
          pltpu.VMEM((CHUNK * DIM_,), jnp.float32),
          pltpu.VMEM((CHUNK * DIM_,), jnp.float32),
          pltpu.VMEM((CHUNK * DIM_,), jnp.float32),
          pltpu.VMEM((CHUNK * DIM_,), jnp.float32),
          pltpu.VMEM((CHUNK * DIM_,), jnp.float32),
          pltpu.VMEM((LANES,), jnp.float32),
          pltpu.SemaphoreType.DMA,
      ],
  )
  return compute(rows)


def kernel(u_pos, v_pos, v_neg, src_w, tgt_w):
  u_pos = u_pos.astype(jnp.int32)
  v_pos = v_pos.astype(jnp.int32)
  v_neg_t = v_neg.astype(jnp.int32).T
  stail = src_w[TAIL:, :]
  ttail = tgt_w[TAIL:, :]
  partials = _sc_call(u_pos, v_pos,
                      v_neg_t[0], v_neg_t[1], v_neg_t[2], v_neg_t[3],
                      v_neg_t[4], src_w.T, tgt_w.T, stail, ttail)
  return partials.sum() / jnp.float32(B_ * NNEG_)


# speedup vs baseline: 1.0742x; 1.0742x over previous
import functools

import jax
import jax.numpy as jnp
from jax import lax
from jax.experimental import pallas as pl
from jax.experimental.pallas import tpu as pltpu
from jax.experimental.pallas import tpu_sc as plsc

VOCAB_ = 1000000
DIM_ = 64
B_ = 16384
NNEG_ = 5
MARGIN_ = 1.0

NC = 2
NS = 16
NW = NC * NS
LANES = 16

NPOS = 7 * B_
SB = 512
NSB = 61
WRANGE = NSB * SB
TAIL = 7812 * 128
ECAP = 6144
NG = 4
GSPAN = 8192

CHUNK = 128
GROUPS = CHUNK // LANES
BPW = B_ // NW


def _shuf(x, perm):
  return lax.gather(
      x, perm[:, None],
      lax.GatherDimensionNumbers(offset_dims=(), collapsed_slice_dims=(0,),
                                 start_index_map=(0,)),
      slice_sizes=(1,), mode=lax.GatherScatterMode.PROMISE_IN_BOUNDS)


def _gather_body(u_hbm, v_hbm, n0_hbm, n1_hbm, n2_hbm, n3_hbm, n4_hbm,
                 srct_hbm, tgtt_hbm, stail_hbm, ttail_hbm, rows_hbm,
                 blk, tblk, ev, ep, gv, gp, sv, sp, ixb, rowstage,
                 semb, semr):
  cid = lax.axis_index("c")
  sid = lax.axis_index("s")
  wid = cid * NS + sid
  lane = lax.iota(jnp.int32, LANES)

  lo = wid * WRANGE
  is_last = wid == NW - 1
  hi = jnp.where(is_last, VOCAB_, lo + WRANGE)
  nsb = jnp.where(is_last, NSB + 1, NSB)

  def sweep(tab_hbm, tail_hbm, idx_arrays, pos_base_list):
    def scan_round(r):
      def scan_arr(carry, arr, pbase):
        cur, seen = carry

        def chunk_body(c, carry2):
          cur2, seen2 = carry2
          pltpu.sync_copy(arr.at[pl.ds(c * 2048, 2048)], ixb)

          def vec_body(k, carry3):
            cur3, seen3 = carry3
            v = ixb[pl.ds(k * LANES, LANES)]
            m = (v >= lo) & (v < hi)
            call = plsc.all_reduce_population_count(m)[0]

            def append(args):
              cur4, seen4 = args
              mi = m.astype(jnp.int32)
              pref = plsc.cumsum(mi)
              rank = seen4 + pref - 1
              rlo = r * ECAP
              m2 = m & (rank >= rlo) & (rank < rlo + ECAP)
              posv = pbase + c * 2048 + k * LANES + lane
              plsc.store_compressed(ev.at[pl.ds(cur4, LANES)], v, mask=m2)
              plsc.store_compressed(ep.at[pl.ds(cur4, LANES)], posv, mask=m2)
              c2 = plsc.all_reduce_population_count(m2)[0]
              return cur4 + c2, seen4 + call

            return lax.cond(call > 0, append, lambda a: a, (cur3, seen3))

          return lax.fori_loop(0, 128, vec_body, (cur2, seen2),
                               unroll=2)

        return lax.fori_loop(0, B_ // 2048, chunk_body, (cur, seen))

      cur, seen = jnp.int32(0), jnp.int32(0)
      for arr, pbase in zip(idx_arrays, pos_base_list):
        cur, seen = scan_arr((cur, seen), arr, pbase)
      return cur, seen

    def group_pass(nent):
      nvec = (nent + LANES - 1) // LANES
      gcs = []
      for g in range(NG):
        glo = lo + g * GSPAN

        def gp_body(j, gcur, glo=glo, g=g):
          v = ev[pl.ds(j * LANES, LANES)]
          valid = (j * LANES + lane) < nent
          m = valid & (v >= glo) & (v < glo + GSPAN)
          cm = plsc.all_reduce_population_count(m)[0]

          def app(c2):
            p = ep[pl.ds(j * LANES, LANES)]
            plsc.store_compressed(gv.at[pl.ds(g * ECAP + c2, LANES)], v,
                                  mask=m)
            plsc.store_compressed(gp.at[pl.ds(g * ECAP + c2, LANES)], p,
                                  mask=m)
            return c2 + cm

          return lax.cond(cm > 0, app, lambda a: a, gcur)

        gcs.append(lax.fori_loop(0, nvec, gp_body, jnp.int32(0)))
      return gcs

    def extract_block(width, blo, gcnt, gbase, src_sel):
      src_buf = (blk, tblk)[src_sel]
      def cmp_body(j, scur):
        v = gv[pl.ds(gbase + j * LANES, LANES)]
        valid = (j * LANES + lane) < gcnt
        m = valid & (v >= blo) & (v < blo + width)
        cm = plsc.all_reduce_population_count(m)[0]

        def append(scur2):
          p = gp[pl.ds(gbase + j * LANES, LANES)]
          plsc.store_compressed(sv.at[pl.ds(scur2, LANES)], v, mask=m)
          plsc.store_compressed(sp.at[pl.ds(scur2, LANES)], p, mask=m)
          return scur2 + cm

        return lax.cond(cm > 0, append, lambda a: a, scur)

      nvec = (gcnt + LANES - 1) // LANES
      scnt = lax.fori_loop(0, nvec, cmp_body, jnp.int32(0))

      def ext_body(jj, carry):
        v16 = sv[pl.ds(jj * LANES, LANES)]
        p16 = sp[pl.ds(jj * LANES, LANES)]
        em = (jj * LANES + lane) < scnt
        col = jnp.where(em, v16 - blo, 0)
        posd = jnp.where(em, p16, NPOS + lane)
        for d in range(DIM_):
          if src_sel == 1:
            g = plsc.load_gather(src_buf,
                                 [col, jnp.full((LANES,), d, jnp.int32)])
          else:
            g = plsc.load_gather(src_buf,
                                 [jnp.full((LANES,), d, jnp.int32), col])
          plsc.store_scatter(rowstage, [lane * DIM_ + d], g)
        for j in range(LANES):
          pltpu.make_async_copy(
              rowstage.at[pl.ds(j * DIM_, DIM_)],
              rows_hbm.at[pl.ds(posd[j] * DIM_, DIM_)], semr).start()
        for j in range(LANES):
          pltpu.make_async_copy(rows_hbm.at[pl.ds(0, DIM_)],
                                rowstage.at[pl.ds(0, DIM_)], semr).wait()
        return carry

      nsvec = (scnt + LANES - 1) // LANES
      lax.fori_loop(0, nsvec, ext_body, jnp.int32(0))

    def round_body(carry):
      r, _total = carry
      nent, total = scan_round(r)
      gcs = group_pass(nent)

      def blk_body(i, c2):
        blo = lo + i * SB
        g = i // (GSPAN // SB)
        gbase = g * ECAP
        gcnt = jnp.where(g == 0, gcs[0],
                         jnp.where(g == 1, gcs[1],
                                   jnp.where(g == 2, gcs[2], gcs[3])))
        pltpu.sync_copy(tab_hbm.at[:, pl.ds(blo, SB)], blk)
        extract_block(SB, blo, gcnt, gbase, 0)
        return c2

      lax.fori_loop(0, nsb, blk_body, jnp.int32(0))

      @pl.when(is_last)
      def _():
        pltpu.sync_copy(tail_hbm, tblk)
        extract_block(VOCAB_ - TAIL, jnp.int32(TAIL), gcs[NG - 1],
                      (NG - 1) * ECAP, 1)

      return r + 1, total

    def round_cond(carry):
      r, total = carry
      return jnp.logical_or(r == 0, r * ECAP < total)

    lax.while_loop(round_cond, round_body, (jnp.int32(0), jnp.int32(0)))

  sweep(srct_hbm, stail_hbm, [u_hbm], [0])
  sweep(tgtt_hbm, ttail_hbm,
        [v_hbm, n0_hbm, n1_hbm, n2_hbm, n3_hbm, n4_hbm],
        [B_, 2 * B_, 3 * B_, 4 * B_, 5 * B_, 6 * B_])


def _compute_body(rows_hbm, out_hbm, ru, rv, rn0, rn1, rn2, rn3, rn4,
                  acc_v, sem):
  cid = lax.axis_index("c")
  sid = lax.axis_index("s")
  wid = cid * NS + sid
  lane = lax.iota(jnp.int32, LANES)
  perms = [lane ^ 1, lane ^ 2, lane ^ 4, lane ^ 8]
  total = jnp.zeros((LANES,), jnp.float32)

  for chunk in range(BPW // CHUNK):
    base = wid * BPW + chunk * CHUNK
    bufs = (ru, rv, rn0, rn1, rn2, rn3, rn4)
    cps = [pltpu.make_async_copy(
        rows_hbm.at[pl.ds((t * B_ + base) * DIM_, CHUNK * DIM_)], bufs[t], sem)
        for t in range(7)]
    for cp in cps:
      cp.start()
    for cp in cps:
      cp.wait()

    def elem_body(e, tot):
      pv = jnp.zeros((LANES,), jnp.float32)
      p0 = jnp.zeros((LANES,), jnp.float32)
      p1 = jnp.zeros((LANES,), jnp.float32)
      p2 = jnp.zeros((LANES,), jnp.float32)
      p3 = jnp.zeros((LANES,), jnp.float32)
      p4 = jnp.zeros((LANES,), jnp.float32)
      for k in range(DIM_ // LANES):
        sl = pl.ds(e * DIM_ + k * LANES, LANES)
        uc = ru[sl]
        pv = pv + uc * rv[sl]
        p0 = p0 + uc * rn0[sl]
        p1 = p1 + uc * rn1[sl]
        p2 = p2 + uc * rn2[sl]
        p3 = p3 + uc * rn3[sl]
        p4 = p4 + uc * rn4[sl]
      loss = jnp.zeros((LANES,), jnp.float32)
      for p in (p0, p1, p2, p3, p4):
        r = p - pv
        for perm in perms:
          r = r + _shuf(r, perm)
        loss = loss + jnp.maximum(r + MARGIN_, 0.0)
      return tot + loss

    total = total + lax.fori_loop(0, CHUNK, elem_body,
                                  jnp.zeros((LANES,), jnp.float32))

  acc_v[...] = jnp.where(lane == 0, total, jnp.float32(0.0))
  pltpu.sync_copy(acc_v, out_hbm.at[pl.ds(wid * LANES, LANES)])


@jax.jit
def _sc_call(u_pos, v_pos, n0, n1, n2, n3, n4, src_t, tgt_t, stail, ttail):
  mesh = plsc.VectorSubcoreMesh(core_axis_name="c", subcore_axis_name="s")
  cp = pltpu.CompilerParams(needs_layout_passes=False)
  gather = pl.kernel(
      _gather_body,
      out_type=jax.ShapeDtypeStruct(((NPOS + LANES) * DIM_,), jnp.float32),
      mesh=mesh,
      compiler_params=cp,
      scratch_types=[
          pltpu.VMEM((DIM_, SB), jnp.float32),
          pltpu.VMEM((VOCAB_ - TAIL, DIM_), jnp.float32),
          pltpu.VMEM((ECAP + LANES, ), jnp.int32),
          pltpu.VMEM((ECAP + LANES, ), jnp.int32),
          pltpu.VMEM((NG * ECAP + LANES, ), jnp.int32),
          pltpu.VMEM((NG * ECAP + LANES, ), jnp.int32),
          pltpu.VMEM((ECAP + LANES, ), jnp.int32),
          pltpu.VMEM((ECAP + LANES, ), jnp.int32),
          pltpu.VMEM((2048,), jnp.int32),
          pltpu.VMEM((LANES * DIM_,), jnp.float32),
          pltpu.SemaphoreType.DMA,
          pltpu.SemaphoreType.DMA,
      ],
  )
  rows = gather(u_pos, v_pos, n0, n1, n2, n3, n4, src_t, tgt_t, stail, ttail)
  compute = pl.kernel(
      _compute_body,
      out_type=jax.ShapeDtypeStruct((NW * LANES,), jnp.float32),
      mesh=mesh,
      compiler_params=cp,
      scratch_types=[
          pltpu.VMEM((CHUNK * DIM_,), jnp.float32),
          pltpu.VMEM((CHUNK * DIM_,), jnp.float32),
          pltpu.VMEM((CHUNK * DIM_,), jnp.float32),
          pltpu.VMEM((CHUNK * DIM_,), jnp.float32),
          pltpu.VMEM((CHUNK * DIM_,), jnp.float32),
          pltpu.VMEM((CHUNK * DIM_,), jnp.float32),
          pltpu.VMEM((CHUNK * DIM_,), jnp.float32),
          pltpu.VMEM((LANES,), jnp.float32),
          pltpu.SemaphoreType.DMA,
      ],
  )
  return compute(rows)


def kernel(u_pos, v_pos, v_neg, src_w, tgt_w):
  u_pos = u_pos.astype(jnp.int32)
  v_pos = v_pos.astype(jnp.int32)
  v_neg_t = v_neg.astype(jnp.int32).T
  stail = src_w[TAIL:, :]
  ttail = tgt_w[TAIL:, :]
  partials = _sc_call(u_pos, v_pos,
                      v_neg_t[0], v_neg_t[1], v_neg_t[2], v_neg_t[3],
                      v_neg_t[4], src_w.T, tgt_w.T, stail, ttail)
  return partials.sum() / jnp.float32(B_ * NNEG_)


# speedup vs baseline: 1.2632x; 1.1760x over previous
import functools

import jax
import jax.numpy as jnp
from jax import lax
from jax.experimental import pallas as pl
from jax.experimental.pallas import tpu as pltpu
from jax.experimental.pallas import tpu_sc as plsc

VOCAB_ = 1000000
DIM_ = 64
B_ = 16384
NNEG_ = 5
MARGIN_ = 1.0

NC = 2
NS = 16
NW = NC * NS
LANES = 16

BPW = B_ // NW
CHUNK = 128
NCHUNK = BPW // CHUNK
GROUPS = CHUNK // LANES


def _shuf(x, perm):
  return lax.gather(
      x, perm[:, None],
      lax.GatherDimensionNumbers(offset_dims=(), collapsed_slice_dims=(0,),
                                 start_index_map=(0,)),
      slice_sizes=(1,), mode=lax.GatherScatterMode.PROMISE_IN_BOUNDS)


def _sc_body(u_hbm, v_hbm, n0_hbm, n1_hbm, n2_hbm, n3_hbm, n4_hbm,
             src_hbm, tgt_hbm, out_hbm,
             iu, iv, in0, in1, in2, in3, in4,
             ru, rv, rn0, rn1, rn2, rn3, rn4,
             acc_v, sem):
  cid = lax.axis_index("c")
  sid = lax.axis_index("s")
  wid = cid * NS + sid

  lane = lax.iota(jnp.int32, LANES)
  perms = [lane ^ 1, lane ^ 2, lane ^ 4, lane ^ 8]
  total = jnp.zeros((LANES,), jnp.float32)

  for chunk in range(NCHUNK):
    base = wid * BPW + chunk * CHUNK
    pltpu.sync_copy(u_hbm.at[pl.ds(base, CHUNK)], iu)
    pltpu.sync_copy(v_hbm.at[pl.ds(base, CHUNK)], iv)
    pltpu.sync_copy(n0_hbm.at[pl.ds(base, CHUNK)], in0)
    pltpu.sync_copy(n1_hbm.at[pl.ds(base, CHUNK)], in1)
    pltpu.sync_copy(n2_hbm.at[pl.ds(base, CHUNK)], in2)
    pltpu.sync_copy(n3_hbm.at[pl.ds(base, CHUNK)], in3)
    pltpu.sync_copy(n4_hbm.at[pl.ds(base, CHUNK)], in4)

    def row_dma(g, carry):
      gbase = g * LANES
      sl = pl.ds(gbase, LANES)
      vu, vv = iu[sl], iv[sl]
      v0, v1, v2, v3, v4 = in0[sl], in1[sl], in2[sl], in3[sl], in4[sl]
      for j in range(LANES):
        e = gbase + j
        pltpu.make_async_copy(src_hbm.at[vu[j]], ru.at[e], sem).start()
        pltpu.make_async_copy(tgt_hbm.at[vv[j]], rv.at[e], sem).start()
        pltpu.make_async_copy(tgt_hbm.at[v0[j]], rn0.at[e], sem).start()
        pltpu.make_async_copy(tgt_hbm.at[v1[j]], rn1.at[e], sem).start()
        pltpu.make_async_copy(tgt_hbm.at[v2[j]], rn2.at[e], sem).start()
        pltpu.make_async_copy(tgt_hbm.at[v3[j]], rn3.at[e], sem).start()
        pltpu.make_async_copy(tgt_hbm.at[v4[j]], rn4.at[e], sem).start()
      return carry

    lax.fori_loop(0, GROUPS, row_dma, jnp.int32(0))
    for buf in (ru, rv, rn0, rn1, rn2, rn3, rn4):
      pltpu.make_async_copy(src_hbm.at[pl.ds(0, CHUNK)], buf, sem).wait()

    def elem_body(e, tot):
      pv = jnp.zeros((LANES,), jnp.float32)
      p0 = jnp.zeros((LANES,), jnp.float32)
      p1 = jnp.zeros((LANES,), jnp.float32)
      p2 = jnp.zeros((LANES,), jnp.float32)
      p3 = jnp.zeros((LANES,), jnp.float32)
      p4 = jnp.zeros((LANES,), jnp.float32)
      for k in range(DIM_ // LANES):
        sl = pl.ds(k * LANES, LANES)
        uc = ru[e, sl]
        pv = pv + uc * rv[e, sl]
        p0 = p0 + uc * rn0[e, sl]
        p1 = p1 + uc * rn1[e, sl]
        p2 = p2 + uc * rn2[e, sl]
        p3 = p3 + uc * rn3[e, sl]
        p4 = p4 + uc * rn4[e, sl]
      loss = jnp.zeros((LANES,), jnp.float32)
      for p in (p0, p1, p2, p3, p4):
        r = p - pv
        for perm in perms:
          r = r + _shuf(r, perm)
        loss = loss + jnp.maximum(r + MARGIN_, 0.0)
      return tot + loss

    total = total + lax.fori_loop(0, CHUNK, elem_body,
                                  jnp.zeros((LANES,), jnp.float32))

  acc_v[...] = jnp.where(lane == 0, total, jnp.float32(0.0))
  pltpu.sync_copy(acc_v, out_hbm.at[pl.ds(wid * LANES, LANES)])


@jax.jit
def _sc_call(u_pos, v_pos, n0, n1, n2, n3, n4, src_w, tgt_w):
  mesh = plsc.VectorSubcoreMesh(core_axis_name="c", subcore_axis_name="s")
  f = pl.kernel(
      _sc_body,
      out_type=jax.ShapeDtypeStruct((NW * LANES,), jnp.float32),
      mesh=mesh,
      scratch_types=[
          pltpu.VMEM((CHUNK,), jnp.int32),
          pltpu.VMEM((CHUNK,), jnp.int32),
          pltpu.VMEM((CHUNK,), jnp.int32),
          pltpu.VMEM((CHUNK,), jnp.int32),
          pltpu.VMEM((CHUNK,), jnp.int32),
          pltpu.VMEM((CHUNK,), jnp.int32),
          pltpu.VMEM((CHUNK,), jnp.int32),
          pltpu.VMEM((CHUNK, DIM_), jnp.float32),
          pltpu.VMEM((CHUNK, DIM_), jnp.float32),
          pltpu.VMEM((CHUNK, DIM_), jnp.float32),
          pltpu.VMEM((CHUNK, DIM_), jnp.float32),
          pltpu.VMEM((CHUNK, DIM_), jnp.float32),
          pltpu.VMEM((CHUNK, DIM_), jnp.float32),
          pltpu.VMEM((CHUNK, DIM_), jnp.float32),
          pltpu.VMEM((LANES,), jnp.float32),
          pltpu.SemaphoreType.DMA,
      ],
  )
  return f(u_pos, v_pos, n0, n1, n2, n3, n4, src_w, tgt_w)


def kernel(u_pos, v_pos, v_neg, src_w, tgt_w):
  u_pos = u_pos.astype(jnp.int32)
  v_pos = v_pos.astype(jnp.int32)
  v_neg_t = v_neg.astype(jnp.int32).T
  partials = _sc_call(u_pos, v_pos,
                      v_neg_t[0], v_neg_t[1], v_neg_t[2], v_neg_t[3],
                      v_neg_t[4], src_w, tgt_w)
  return partials.sum() / jnp.float32(B_ * NNEG_)
